# baseline (speedup 1.0000x reference)
import functools

import jax
import jax.numpy as jnp
from jax import lax
from jax.experimental import pallas as pl
from jax.experimental.pallas import tpu as pltpu
from jax.experimental.pallas import tpu_sc as plsc

_NUM_EXPERTS = 64
_HIDDEN = 1024
_TOP_K = 2
_BT = 4096
_NC, _NS, _L = 2, 16, 16


def _matmul_body(x_ref, w_ref, logits_ref):
    logits_ref[...] = lax.dot_general(
        x_ref[...], w_ref[...], (((1,), (1,)), ((), ())),
        preferred_element_type=jnp.float32,
    )


def _tc_logits(hidden_states, W):
    T, H = hidden_states.shape
    E = W.shape[0]
    return pl.pallas_call(
        _matmul_body,
        grid=(T // _BT,),
        in_specs=[
            pl.BlockSpec((_BT, H), lambda i: (i, 0)),
            pl.BlockSpec((E, H), lambda i: (0, 0)),
        ],
        out_specs=pl.BlockSpec((_BT, E), lambda i: (i, 0)),
        out_shape=jax.ShapeDtypeStruct((T, E), jnp.float32),
    )(hidden_states, W)


def _sc_topk(logits):
    T, E = logits.shape
    nw = _NC * _NS
    ntok = T // nw
    ngrp = ntok // _L

    mesh = plsc.VectorSubcoreMesh(core_axis_name="c", subcore_axis_name="s")

    @functools.partial(
        pl.kernel,
        out_type=[
            jax.ShapeDtypeStruct((T * _TOP_K,), jnp.float32),
            jax.ShapeDtypeStruct((T * _TOP_K,), jnp.int32),
        ],
        mesh=mesh,
        compiler_params=pltpu.CompilerParams(needs_layout_passes=False),
        scratch_types=[
            pltpu.VMEM((ntok * E,), jnp.float32),
            pltpu.VMEM((ntok * _TOP_K,), jnp.float32),
            pltpu.VMEM((ntok * _TOP_K,), jnp.int32),
        ],
    )
    def sc_kernel(logits_hbm, wout_hbm, iout_hbm, lg_v, wv, iv):
        wid = lax.axis_index("s") * _NC + lax.axis_index("c")
        base = wid * ntok
        pltpu.sync_copy(logits_hbm.at[pl.ds(base * E, ntok * E)], lg_v)

        lane = lax.iota(jnp.int32, _L)
        neg_inf = jnp.full((_L,), -jnp.inf, jnp.float32)
        zero_i = jnp.zeros((_L,), jnp.int32)

        def group(g, carry):
            tok = g * _L + lane
            row0 = tok * E
            m1, m2 = neg_inf, neg_inf
            i1, i2 = zero_i, zero_i
            for e in range(E):
                col = jnp.full((_L,), e, jnp.int32)
                v = plsc.load_gather(lg_v, [row0 + e])
                gt1 = v > m1
                gt2 = v > m2
                m2 = jnp.where(gt1, m1, jnp.where(gt2, v, m2))
                i2 = jnp.where(gt1, i1, jnp.where(gt2, col, i2))
                m1 = jnp.where(gt1, v, m1)
                i1 = jnp.where(gt1, col, i1)
            e2 = jnp.exp(m2 - m1)
            w1 = 1.0 / (1.0 + e2)
            w2 = 1.0 - w1
            out0 = tok * _TOP_K
            plsc.store_scatter(wv, [out0], w1)
            plsc.store_scatter(wv, [out0 + 1], w2)
            plsc.store_scatter(iv, [out0], i1)
            plsc.store_scatter(iv, [out0 + 1], i2)
            return carry

        lax.fori_loop(0, ngrp, group, 0)
        pltpu.sync_copy(wv, wout_hbm.at[pl.ds(base * _TOP_K, ntok * _TOP_K)])
        pltpu.sync_copy(iv, iout_hbm.at[pl.ds(base * _TOP_K, ntok * _TOP_K)])

    return sc_kernel(logits.reshape(T * E))


def kernel(hidden_states, W):
    logits = _tc_logits(hidden_states, W)
    wout, iout = _sc_topk(logits)
    T = hidden_states.shape[0]
    return (wout.reshape(T, _TOP_K), logits, iout.reshape(T, _TOP_K))
